# baseline (speedup 1.0000x reference)
import jax
import jax.numpy as jnp
from jax.experimental import pallas as pl
from jax.experimental.pallas import tpu as pltpu

_EPS = 1e-06
_HD = 512
_WD = 512


def _interp_body(nbr_ref, cam_ref, d_ref, o_ref, acc_ref):
    i = pl.program_id(0)
    k = pl.num_programs(0)

    c0 = cam_ref[0]
    c1 = cam_ref[1]

    def _pre(j):
        t = jnp.abs((c0 - nbr_ref[j, 0].astype(jnp.float32))
                    * (c1 - nbr_ref[j, 1].astype(jnp.float32)))
        return jnp.where(t <= _EPS, 0.0, t)

    pres = [_pre(j) for j in range(8)]
    s = pres[0]
    for j in range(1, 8):
        s = s + pres[j]
    flip = 7 - i
    w_pre = jnp.float32(0.0)
    for j in range(8):
        w_pre = jnp.where(flip == j, pres[j], w_pre)
    w = w_pre / s
    w = jnp.where(jnp.abs(w) <= _EPS, 0.0, w)

    n0 = nbr_ref[i, 0]
    n1 = nbr_ref[i, 1]
    rolled = pltpu.roll(d_ref[0], (_HD - n0) % _HD, axis=0)
    rolled = pltpu.roll(rolled, (3 * _WD - 3 * n1) % (3 * _WD), axis=1)
    contrib = w * rolled

    @pl.when(i == 0)
    def _():
        acc_ref[...] = contrib

    @pl.when(i > 0)
    def _():
        acc_ref[...] = acc_ref[...] + contrib

    @pl.when(i == k - 1)
    def _():
        t = acc_ref[...]
        td = jnp.concatenate([t, t], axis=0)
        tdd = jnp.concatenate([td, td[:, :_HD]], axis=1)
        wstart = (0, 2 * _HD, _HD)
        for c in range(3):
            r0 = c * 341
            ws = [tdd[r0:r0 + 342, wstart[ph]:wstart[ph] + 1024]
                  for ph in range(3)]
            v = jnp.stack(ws, axis=1).reshape(1026, 1024)
            o_ref[c:c + 1] = v[c:c + 1024][None]


def kernel(pixel, cam_xyz, neighbors, data):
    H, W = pixel.shape
    nbr = neighbors.astype(jnp.int32)
    data_r = data.reshape(64, _HD, _WD, 3)
    idx = nbr[:, 0] * 8 + nbr[:, 1]
    sel = jnp.concatenate(
        [jax.lax.dynamic_slice(data_r, (idx[i], 0, 0, 0), (1, _HD, _WD, 3))
         for i in range(8)], axis=0)
    d2 = sel.reshape(8, _HD, 3 * _WD)
    camxy = cam_xyz[:2].astype(jnp.float32)

    grid_spec = pltpu.PrefetchScalarGridSpec(
        num_scalar_prefetch=2,
        grid=(8,),
        in_specs=[
            pl.BlockSpec(
                (1, _HD, 3 * _WD),
                lambda i, nref, cref: (i, 0, 0),
            ),
        ],
        out_specs=pl.BlockSpec((3, 2 * _HD, 2 * _WD),
                               lambda i, nref, cref: (0, 0, 0)),
        scratch_shapes=[pltpu.VMEM((_HD, 3 * _WD), jnp.float32)],
    )

    return pl.pallas_call(
        _interp_body,
        grid_spec=grid_spec,
        out_shape=jax.ShapeDtypeStruct((3, 2 * _HD, 2 * _WD), jnp.float32),
    )(nbr, camxy, d2)


# speedup vs baseline: 3.2847x; 3.2847x over previous
import jax
import jax.numpy as jnp
from jax.experimental import pallas as pl
from jax.experimental.pallas import tpu as pltpu

_EPS = 1e-06
_HD = 512
_WD = 512


def _interp_body(nbr_ref, cam_ref, d_ref, o_ref, acc_ref):
    i = pl.program_id(0)
    k = pl.num_programs(0)

    c0 = cam_ref[0]
    c1 = cam_ref[1]

    def _pre(j):
        t = jnp.abs((c0 - nbr_ref[j, 0].astype(jnp.float32))
                    * (c1 - nbr_ref[j, 1].astype(jnp.float32)))
        return jnp.where(t <= _EPS, 0.0, t)

    pres = [_pre(j) for j in range(8)]
    s = pres[0]
    for j in range(1, 8):
        s = s + pres[j]
    flip = 7 - i
    w_pre = jnp.float32(0.0)
    for j in range(8):
        w_pre = jnp.where(flip == j, pres[j], w_pre)
    w = w_pre / s
    w = jnp.where(jnp.abs(w) <= _EPS, 0.0, w)

    n0 = nbr_ref[i, 0]
    n1 = nbr_ref[i, 1]
    rolled = pltpu.roll(d_ref[0], (_HD - n0) % _HD, axis=0)
    rolled = pltpu.roll(rolled, (3 * _WD - 3 * n1) % (3 * _WD), axis=1)
    contrib = w * rolled

    @pl.when(i == 0)
    def _():
        acc_ref[...] = contrib

    @pl.when(i > 0)
    def _():
        acc_ref[...] = acc_ref[...] + contrib

    @pl.when(i == k - 1)
    def _():
        t = acc_ref[...]
        td = jnp.concatenate([t, t], axis=0)
        tdd = jnp.concatenate([td, td[:, :_HD]], axis=1)
        wstart = (0, 2 * _HD, _HD)
        for c in range(3):
            r0 = c * 341
            ws = [tdd[r0:r0 + 342, wstart[ph]:wstart[ph] + 1024]
                  for ph in range(3)]
            v = jnp.stack(ws, axis=1).reshape(1026, 1024)
            o_ref[c:c + 1] = v[c:c + 1024][None]


def kernel(pixel, cam_xyz, neighbors, data):
    H, W = pixel.shape
    nbr = neighbors.astype(jnp.int32)
    sel = jnp.take(data.reshape(64, _HD, _WD, 3), nbr[:, 0] * 8 + nbr[:, 1],
                   axis=0, mode='clip')
    d2 = sel.reshape(8, _HD, 3 * _WD)
    camxy = cam_xyz[:2].astype(jnp.float32)

    grid_spec = pltpu.PrefetchScalarGridSpec(
        num_scalar_prefetch=2,
        grid=(8,),
        in_specs=[
            pl.BlockSpec(
                (1, _HD, 3 * _WD),
                lambda i, nref, cref: (i, 0, 0),
            ),
        ],
        out_specs=pl.BlockSpec((3, 2 * _HD, 2 * _WD),
                               lambda i, nref, cref: (0, 0, 0)),
        scratch_shapes=[pltpu.VMEM((_HD, 3 * _WD), jnp.float32)],
    )

    return pl.pallas_call(
        _interp_body,
        grid_spec=grid_spec,
        out_shape=jax.ShapeDtypeStruct((3, 2 * _HD, 2 * _WD), jnp.float32),
    )(nbr, camxy, d2)
